# baseline (speedup 1.0000x reference)
import functools

import jax
import jax.numpy as jnp
from jax import lax
from jax.experimental import pallas as pl
from jax.experimental.pallas import tpu as pltpu

_NBUF = 4
_NIN = 4


def _expand_matrix(w, s):
    p = jnp.arange(w, dtype=jnp.int32)
    q = jnp.arange(s * s * w, dtype=jnp.int32)
    return ((q[None, :] % (s * w)) // s == p[:, None]).astype(jnp.float32)


def _up_body(r_ref, xr_hbm, xi_hbm, o_hbm, xbuf, isem, obuf, osem,
             *, br, nrow):
    i = pl.program_id(0)

    def in_copy(s_, step):
        row = step * br
        return (pltpu.make_async_copy(xr_hbm.at[pl.ds(row, br), :],
                                      xbuf.at[s_, 0], isem.at[s_, 0]),
                pltpu.make_async_copy(xi_hbm.at[pl.ds(row, br), :],
                                      xbuf.at[s_, 1], isem.at[s_, 1]))

    def out_copy(s_, row):
        return (pltpu.make_async_copy(obuf.at[s_, 0],
                                      o_hbm.at[0, pl.ds(row, br), :],
                                      osem.at[s_, 0]),
                pltpu.make_async_copy(obuf.at[s_, 1],
                                      o_hbm.at[1, pl.ds(row, br), :],
                                      osem.at[s_, 1]))

    look = _NIN - 1

    @pl.when(i == 0)
    def _():
        for k in range(min(look, nrow)):
            for cp in in_copy(k % _NIN, k):
                cp.start()

    @pl.when(i + look < nrow)
    def _():
        for cp in in_copy((i + look) % _NIN, i + look):
            cp.start()

    cur = lax.rem(i, _NIN)
    for cp in in_copy(cur, 0):
        cp.wait()

    slot = lax.rem(i, _NBUF)

    @pl.when(i >= _NBUF)
    def _():
        for cp in out_copy(slot, 0):
            cp.wait()

    r = r_ref[...]
    obuf[slot, 0] = jnp.dot(xbuf[cur, 0], r,
                            preferred_element_type=jnp.float32)
    obuf[slot, 1] = jnp.dot(xbuf[cur, 1], r,
                            preferred_element_type=jnp.float32)
    for cp in out_copy(slot, i * br):
        cp.start()

    @pl.when(i == nrow - 1)
    def _():
        for k in range(min(nrow, _NBUF)):
            for cp in out_copy((nrow - 1 - k) % _NBUF, 0):
                cp.wait()


@functools.partial(jax.jit, static_argnames=())
def kernel(xr, xi):
    n, c, h, w = xr.shape
    s = 2
    t = n * c * h
    m = s * s * w

    br = 1024
    while t % br:
        br //= 2
    nrow = t // br

    r = _expand_matrix(w, s)
    xr2 = xr.reshape(t, w)
    xi2 = xi.reshape(t, w)

    out = pl.pallas_call(
        functools.partial(_up_body, br=br, nrow=nrow),
        out_shape=jax.ShapeDtypeStruct((2, t, m), jnp.float32),
        grid=(nrow,),
        in_specs=[
            pl.BlockSpec((w, m), lambda i: (0, 0)),
            pl.BlockSpec(memory_space=pl.ANY),
            pl.BlockSpec(memory_space=pl.ANY),
        ],
        out_specs=pl.BlockSpec(memory_space=pl.ANY),
        scratch_shapes=[
            pltpu.VMEM((_NIN, 2, br, w), jnp.float32),
            pltpu.SemaphoreType.DMA((_NIN, 2)),
            pltpu.VMEM((_NBUF, 2, br, m), jnp.float32),
            pltpu.SemaphoreType.DMA((_NBUF, 2)),
        ],
        compiler_params=pltpu.CompilerParams(
            dimension_semantics=("arbitrary",)),
        cost_estimate=pl.CostEstimate(
            flops=2 * 2 * t * w * m,
            transcendentals=0,
            bytes_accessed=4 * (2 * t * w + 2 * t * m + w * m)),
    )(r, xr2, xi2)

    return out.reshape(2, n, c, h * s, s * w)


# speedup vs baseline: 1.0107x; 1.0107x over previous
import functools

import jax
import jax.numpy as jnp
from jax.experimental import pallas as pl
from jax.experimental.pallas import tpu as pltpu


def _expand_matrix(w, s):
    p = jnp.arange(w, dtype=jnp.int32)
    q = jnp.arange(s * s * w, dtype=jnp.int32)
    return ((q[None, :] % (s * w)) // s == p[:, None]).astype(jnp.float32)


def _up_body(r_ref, xr_ref, xi_ref, o_ref):
    r = r_ref[...]
    o_ref[0] = jnp.dot(xr_ref[...], r, preferred_element_type=jnp.float32)
    o_ref[1] = jnp.dot(xi_ref[...], r, preferred_element_type=jnp.float32)


@functools.partial(jax.jit, static_argnames=())
def kernel(xr, xi):
    n, c, h, w = xr.shape
    s = 2
    t = n * c * h
    m = s * s * w

    br = 4096
    while t % br:
        br //= 2
    grid = t // br

    r = _expand_matrix(w, s)
    xr2 = xr.reshape(t, w)
    xi2 = xi.reshape(t, w)

    out = pl.pallas_call(
        _up_body,
        out_shape=jax.ShapeDtypeStruct((2, t, m), jnp.float32),
        grid=(grid,),
        in_specs=[
            pl.BlockSpec((w, m), lambda i: (0, 0)),
            pl.BlockSpec((br, w), lambda i: (i, 0)),
            pl.BlockSpec((br, w), lambda i: (i, 0)),
        ],
        out_specs=pl.BlockSpec((2, br, m), lambda i: (0, i, 0)),
        compiler_params=pltpu.CompilerParams(
            dimension_semantics=("arbitrary",)),
        cost_estimate=pl.CostEstimate(
            flops=2 * 2 * t * w * m,
            transcendentals=0,
            bytes_accessed=4 * (2 * t * w + 2 * t * m + w * m)),
    )(r, xr2, xi2)

    return out.reshape(2, n, c, h * s, s * w)
